# Initial kernel scaffold; baseline (speedup 1.0000x reference)
import jax
import jax.numpy as jnp
from jax.experimental import pallas as pl


def kernel(x, edge_index, batch, W1, b1, W2, b2, Wfc, bfc):
    raise NotImplementedError("write your pallas kernel here")


# speedup vs baseline: 2.7399x; 2.7399x over previous
import functools

import jax
import jax.numpy as jnp
from jax import lax
from jax.experimental import pallas as pl
from jax.experimental.pallas import tpu as pltpu
from jax.experimental.pallas import tpu_sc as plsc

_N = 10000
_E = 320000
_DIN = 128
_DH = 64
_G = 64

_NC = 2
_NS = 16
_NW = _NC * _NS
_DP = 128

_K = 125
_EPT = _E // _NW
_NCH = _EPT // _K
_DRPT = 640
_NP = _DRPT * _NS

_NB = 6
_RNG = 1776
_CAP = 10240
_CK = 128
_DUMP = 1791
_PACKDUMP = _DUMP * 16384

_TS = _NB * _CAP
_TRASH = _NW * _TS
_ACC = 1792
_ARPT = _ACC // _NS
_NR = 3
_OUTR = _NB * _ACC


def _deg_body(dst_hbm, out_hbm, dstv, onesv, wb, deg_sh, sem):
    c = lax.axis_index("c")
    s = lax.axis_index("s")
    wid = s * _NC + c

    pltpu.sync_copy(dst_hbm.at[wid], dstv)

    one = jnp.ones((16,), jnp.float32)
    for i in range(_K // 16):
        onesv[pl.ds(i * 16, 16)] = one
    onesv[pl.ds(_K - 16, 16)] = one

    z = jnp.zeros((16,), jnp.float32)

    def zb(i, _):
        wb[pl.ds(i * 16, 16)] = z
        return 0

    lax.fori_loop(0, _DRPT // 16, zb, 0)
    pltpu.sync_copy(wb, deg_sh.at[pl.ds(s * _DRPT, _DRPT)])
    plsc.subcore_barrier()

    def body(j, _):
        pltpu.sync_copy(onesv, deg_sh.at[dstv.at[j]], add=True)
        return 0

    lax.fori_loop(0, _NCH, body, 0)
    plsc.subcore_barrier()

    pltpu.sync_copy(deg_sh.at[pl.ds(s * _DRPT, _DRPT)], wb)
    pltpu.sync_copy(wb, out_hbm.at[pl.ds(c * _NP + s * _DRPT, _DRPT)])


def _bin_body(src_hbm, dst_hbm, pos_hbm, fill_hbm, bins_hbm, srcv, dstv,
              posv, idxb, datb, fidxb, sem):
    c = lax.axis_index("c")
    s = lax.axis_index("s")
    wid = s * _NC + c

    pltpu.sync_copy(src_hbm.at[pl.ds(wid * _EPT, _EPT)], srcv)
    pltpu.sync_copy(dst_hbm.at[pl.ds(wid * _EPT, _EPT)], dstv)
    pltpu.sync_copy(pos_hbm.at[pl.ds(wid * _EPT, _EPT)], posv)
    pltpu.sync_copy(fill_hbm.at[pl.ds(wid * _NB * _CK, _NB * _CK)], fidxb)

    def chunk(j, _):
        for u in range(_CK // 16):
            o = j * _CK + u * 16
            s16 = srcv[pl.ds(o, 16)]
            d16 = dstv[pl.ds(o, 16)]
            b16 = (d16 * 18894) >> 25
            idxb[pl.ds(u * 16, 16)] = posv[pl.ds(o, 16)]
            datb[pl.ds(u * 16, 16)] = (d16 - b16 * _RNG) * 16384 + s16
        pltpu.sync_copy(datb, bins_hbm.at[idxb])
        return 0

    lax.fori_loop(0, _EPT // _CK, chunk, 0)
    o = (_EPT // _CK) * _CK
    ntail = _EPT - o
    for u in range(ntail // 16):
        s16 = srcv[pl.ds(o + u * 16, 16)]
        d16 = dstv[pl.ds(o + u * 16, 16)]
        b16 = (d16 * 18894) >> 25
        idxb[pl.ds(u * 16, 16)] = posv[pl.ds(o + u * 16, 16)]
        datb[pl.ds(u * 16, 16)] = (d16 - b16 * _RNG) * 16384 + s16
    pltpu.sync_copy(datb.at[pl.ds(0, 16)],
                    bins_hbm.at[idxb.at[pl.ds(0, 16)]])

    dump16 = jnp.full((16,), _PACKDUMP, jnp.int32)
    for u in range(_CK // 16):
        datb[pl.ds(u * 16, 16)] = dump16
    def fchunk(j, _):
        for u in range(_CK // 16):
            idxb[pl.ds(u * 16, 16)] = fidxb[pl.ds(j * _CK + u * 16, 16)]
        pltpu.sync_copy(datb, bins_hbm.at[idxb])
        return 0

    lax.fori_loop(0, _NB, fchunk, 0)


def _agg_body(bins_hbm, cnts_hbm, g_hbm, out_hbm, pk, srcb, dstb, rows, zb,
              wb, cbufa, cbufb, agg_sh, sem):
    c = lax.axis_index("c")
    s = lax.axis_index("s")

    z = jnp.zeros((16,), jnp.float32)

    def zz(i, _):
        for t in range(_DP // 16):
            zb[i, pl.ds(t * 16, 16)] = z
        return 0

    lax.fori_loop(0, _ARPT, zz, 0)

    pltpu.sync_copy(cnts_hbm.at[pl.ds((2 * s) * 16, 16)], cbufa)
    pltpu.sync_copy(cnts_hbm.at[pl.ds((2 * s + 1) * 16, 16)], cbufb)


    for r in range(_NR):
        b = 2 * r + c
        pltpu.sync_copy(zb, agg_sh.at[pl.ds(s * _ARPT, _ARPT)])
        plsc.subcore_barrier()

        for t in range(2):
            torig = 2 * s + t
            cref = cbufa if t == 0 else cbufb
            cv = cref[...]
            m = jnp.where(c == 0, cv[2 * r], cv[2 * r + 1])
            segbase = (torig * _NB) * _CAP

            def chunk(j, _):
                pltpu.sync_copy(
                    bins_hbm.at[pl.ds(segbase + b * _CAP + j * _CK, _CK)],
                    pk)
                for u in range(_CK // 16):
                    p = pk[pl.ds(u * 16, 16)]
                    srcb[pl.ds(u * 16, 16)] = p & 16383
                    dstb[pl.ds(u * 16, 16)] = p >> 14
                pltpu.async_copy(g_hbm.at[srcb], rows, sem).wait()
                pltpu.sync_copy(rows, agg_sh.at[dstb], add=True)
                return 0

            lax.fori_loop(0, m, chunk, 0)

        plsc.subcore_barrier()
        pltpu.sync_copy(agg_sh.at[pl.ds(s * _ARPT, _ARPT)], wb)
        pltpu.sync_copy(wb, out_hbm.at[pl.ds(b * _ACC + s * _ARPT, _ARPT)])
        plsc.subcore_barrier()


@functools.cache
def _sc_kernels():
    mesh = plsc.VectorSubcoreMesh(
        core_axis_name="c", subcore_axis_name="s",
        num_cores=_NC, num_subcores=_NS,
    )
    deg = functools.partial(
        pl.kernel,
        out_type=jax.ShapeDtypeStruct((_NC * _NP,), jnp.float32),
        mesh=mesh,
        scratch_types=[
            pltpu.VMEM((_NCH, _K), jnp.int32),
            pltpu.VMEM((_K,), jnp.float32),
            pltpu.VMEM((_DRPT,), jnp.float32),
            pltpu.VMEM_SHARED((_NP,), jnp.float32),
            pltpu.SemaphoreType.DMA,
        ],
    )(_deg_body)
    binf = functools.partial(
        pl.kernel,
        out_type=jax.ShapeDtypeStruct((_NW * _TS + _CK,), jnp.int32),
        mesh=mesh,
        scratch_types=[
            pltpu.VMEM((_EPT,), jnp.int32),
            pltpu.VMEM((_EPT,), jnp.int32),
            pltpu.VMEM((_EPT,), jnp.int32),
            pltpu.VMEM((_CK,), jnp.int32),
            pltpu.VMEM((_CK,), jnp.int32),
            pltpu.VMEM((_NB * _CK,), jnp.int32),
            pltpu.SemaphoreType.DMA,
        ],
    )(_bin_body)
    agg = functools.partial(
        pl.kernel,
        out_type=jax.ShapeDtypeStruct((_OUTR, _DP), jnp.float32),
        mesh=mesh,
        scratch_types=[
            pltpu.VMEM((_CK,), jnp.int32),
            pltpu.VMEM((_CK,), jnp.int32),
            pltpu.VMEM((_CK,), jnp.int32),
            pltpu.VMEM((_CK, _DP), jnp.float32),
            pltpu.VMEM((_ARPT, _DP), jnp.float32),
            pltpu.VMEM((_ARPT, _DP), jnp.float32),
            pltpu.VMEM((16,), jnp.int32),
            pltpu.VMEM((16,), jnp.int32),
            pltpu.VMEM_SHARED((_ACC, _DP), jnp.float32),
            pltpu.SemaphoreType.DMA,
        ],
    )(_agg_body)
    return deg, binf, agg


def _gcn_in(aggp_ref):
    parts = []
    done = 0
    for b in range(_NB):
        n = min(_RNG, _N - done)
        if n > 0:
            parts.append(aggp_ref[b * _ACC:b * _ACC + n, :_DH])
        done += n
    return jnp.concatenate(parts, axis=0)


def _pos_body(dst3_ref, pos_ref, fill_ref, cnts_ref):
    d = dst3_ref[...]
    bkt = (d * 18894) >> 25
    jrow = lax.broadcasted_iota(jnp.int32, (625, 625), 0)
    icol = lax.broadcasted_iota(jnp.int32, (625, 625), 1)
    triv = (jrow < icol).astype(jnp.float32)
    j16 = lax.broadcasted_iota(jnp.int32, (16, 16), 0)
    i16 = lax.broadcasted_iota(jnp.int32, (16, 16), 1)
    tril = (j16 < i16).astype(jnp.float32)
    t = pl.program_id(0)
    bidx16 = lax.broadcasted_iota(jnp.int32, (1, 1, 16), 2)
    kfid = lax.broadcasted_iota(jnp.int32, (1, _CK), 1)
    pos = jnp.zeros((1, 625, 16), jnp.int32)
    cnts = jnp.zeros((1, 1, 16), jnp.int32)
    fills = []
    for b in range(_NB):
        oh = (bkt == b).astype(jnp.float32)
        vcnt = jnp.sum(oh, axis=2)
        vpre = lax.dot_general(vcnt, triv, (((1,), (0,)), ((), ())),
                               preferred_element_type=jnp.float32)
        lpre = lax.dot_general(oh, tril, (((2,), (0,)), ((), ())),
                               preferred_element_type=jnp.float32)
        rank = (vpre[:, :, None] + lpre).astype(jnp.int32)
        pos = pos + jnp.where(bkt == b, b * _CAP + rank, 0)
        tot = (vpre[:, 624] + vcnt[:, 624]).astype(jnp.int32)
        ncb = (tot + _CK - 1) // _CK
        cnts = jnp.where(bidx16 == b, ncb[:, None, None], cnts)
        nfill = ncb * _CK - tot
        fp = jnp.where(kfid < nfill[:, None],
                       b * _CAP + tot[:, None] + kfid + t * _TS,
                       _TRASH)
        fills.append(fp[:, None, :])
    pos_ref[...] = pos + t * _TS
    fill_ref[...] = jnp.concatenate(fills, axis=1)
    cnts_ref[...] = cnts


_pos_call = pl.pallas_call(
    _pos_body,
    grid=(_NW,),
    in_specs=[pl.BlockSpec((1, 625, 16), lambda t: (t, 0, 0))],
    out_specs=[
        pl.BlockSpec((1, 625, 16), lambda t: (t, 0, 0)),
        pl.BlockSpec((1, _NB, _CK), lambda t: (t, 0, 0)),
        pl.BlockSpec((1, 1, 16), lambda t: (t, 0, 0)),
    ],
    out_shape=[
        jax.ShapeDtypeStruct((_NW, 625, 16), jnp.int32),
        jax.ShapeDtypeStruct((_NW, _NB, _CK), jnp.int32),
        jax.ShapeDtypeStruct((_NW, 1, 16), jnp.int32),
    ],
)


def _prep_body(degp_ref, x_ref, w1_ref, g1_ref, dinv_ref):
    deg = degp_ref[0, :_N] + degp_ref[1, :_N] + 1.0
    dinv = lax.rsqrt(deg).reshape(_N, 1)
    h = jnp.dot(x_ref[...], w1_ref[...], preferred_element_type=jnp.float32)
    g1_ref[...] = h * dinv
    dinv_ref[...] = dinv


def _mid_body(aggp_ref, g_ref, dinv_ref, b1_ref, w2_ref, g2_ref):
    agg = _gcn_in(aggp_ref) + g_ref[:, :_DH]
    h = jnp.maximum(agg * dinv_ref[...] + b1_ref[...], 0.0)
    g2_ref[...] = jnp.dot(
        h, w2_ref[...], preferred_element_type=jnp.float32) * dinv_ref[...]


def _fin_body(aggp_ref, g2_ref, dinv_ref, b2_ref, batch_ref, wfc_ref, bfc_ref,
              out_ref):
    agg = _gcn_in(aggp_ref) + g2_ref[:, :_DH]
    h = jnp.maximum(agg * dinv_ref[...] + b2_ref[...], 0.0)
    onehot = (batch_ref[...] == lax.broadcasted_iota(
        jnp.int32, (1, _G), 1)).astype(jnp.float32)
    pooled = lax.dot_general(onehot, h, (((0,), (0,)), ((), ())),
                             preferred_element_type=jnp.float32)
    cnt = lax.dot_general(onehot, jnp.ones((_N, 1), jnp.float32),
                          (((0,), (0,)), ((), ())),
                          preferred_element_type=jnp.float32)
    pooled = pooled / jnp.maximum(cnt, 1.0)
    out_ref[...] = jnp.dot(
        pooled, wfc_ref[...], preferred_element_type=jnp.float32) + bfc_ref[...]


_prep_call = pl.pallas_call(
    _prep_body,
    out_shape=[
        jax.ShapeDtypeStruct((_N, _DP), jnp.float32),
        jax.ShapeDtypeStruct((_N, 1), jnp.float32),
    ],
)

_mid_call = pl.pallas_call(
    _mid_body,
    out_shape=jax.ShapeDtypeStruct((_N, _DP), jnp.float32),
)

_fin_call = pl.pallas_call(
    _fin_body,
    out_shape=jax.ShapeDtypeStruct((_G, 1), jnp.float32),
)


def kernel(x, edge_index, batch, W1, b1, W2, b2, Wfc, bfc):
    _deg_kernel, _bin_kernel, _agg_kernel = _sc_kernels()
    src1d = edge_index[0].reshape(_E)
    dst1d = edge_index[1].reshape(_E)
    dst_deg = edge_index[1].reshape(_NW, _NCH, _K)
    w1p = jnp.pad(W1, ((0, 0), (0, _DP - _DH)))
    w2p = jnp.pad(W2, ((0, 0), (0, _DP - _DH)))

    dst3 = edge_index[1].reshape(_NW, 625, 16)
    post, fill3, cnts2d = _pos_call(dst3)
    pos1d = post.reshape(_E)
    fill1d = fill3.reshape(_NW * _NB * _CK)
    cnts = cnts2d.reshape(_NW * 16)

    degp = _deg_kernel(dst_deg).reshape(_NC, _NP)
    bins = _bin_kernel(src1d, dst1d, pos1d, fill1d)
    g1, dinv = _prep_call(degp, x, w1p)
    aggp1 = _agg_kernel(bins, cnts, g1)
    g2 = _mid_call(aggp1, g1, dinv, b1.reshape(1, _DH), w2p)
    aggp2 = _agg_kernel(bins, cnts, g2)
    out = _fin_call(aggp2, g2, dinv, b2.reshape(1, _DH), batch.reshape(_N, 1),
                    Wfc, bfc.reshape(1, 1))
    return out


# speedup vs baseline: 2.8096x; 1.0254x over previous
import functools

import jax
import jax.numpy as jnp
from jax import lax
from jax.experimental import pallas as pl
from jax.experimental.pallas import tpu as pltpu
from jax.experimental.pallas import tpu_sc as plsc

_N = 10000
_E = 320000
_DIN = 128
_DH = 64
_G = 64

_NC = 2
_NS = 16
_NW = _NC * _NS
_DP = 128

_K = 125
_EPT = _E // _NW
_NCH = _EPT // _K
_DRPT = 640
_NP = _DRPT * _NS

_NB = 6
_RNG = 1776
_CAP = 10240
_CK = 128
_DUMP = 1791
_PACKDUMP = _DUMP * 16384

_TS = _NB * _CAP
_TRASH = _NW * _TS
_ACC = 1792
_ARPT = _ACC // _NS
_NR = 3
_OUTR = _NB * _ACC


def _deg_body(dst_hbm, out_hbm, dstv, onesv, wb, deg_sh, sem):
    c = lax.axis_index("c")
    s = lax.axis_index("s")
    wid = s * _NC + c

    pltpu.sync_copy(dst_hbm.at[wid], dstv)

    one = jnp.ones((16,), jnp.float32)
    for i in range(_K // 16):
        onesv[pl.ds(i * 16, 16)] = one
    onesv[pl.ds(_K - 16, 16)] = one

    z = jnp.zeros((16,), jnp.float32)

    def zb(i, _):
        wb[pl.ds(i * 16, 16)] = z
        return 0

    lax.fori_loop(0, _DRPT // 16, zb, 0)
    pltpu.sync_copy(wb, deg_sh.at[pl.ds(s * _DRPT, _DRPT)])
    plsc.subcore_barrier()

    def body(j, _):
        pltpu.async_copy(onesv, deg_sh.at[dstv.at[j]], sem, add=True)
        return 0

    lax.fori_loop(0, _NCH, body, 0)

    def draindeg(j, _):
        pltpu.make_async_copy(onesv, deg_sh.at[dstv.at[0]], sem).wait()
        return 0

    lax.fori_loop(0, _NCH, draindeg, 0)
    plsc.subcore_barrier()

    pltpu.sync_copy(deg_sh.at[pl.ds(s * _DRPT, _DRPT)], wb)
    pltpu.sync_copy(wb, out_hbm.at[pl.ds(c * _NP + s * _DRPT, _DRPT)])


def _bin_body(src_hbm, dst_hbm, pos_hbm, fill_hbm, bins_hbm, srcv, dstv,
              posv, idxb, datb, fidxb, sem):
    c = lax.axis_index("c")
    s = lax.axis_index("s")
    wid = s * _NC + c

    pltpu.sync_copy(src_hbm.at[pl.ds(wid * _EPT, _EPT)], srcv)
    pltpu.sync_copy(dst_hbm.at[pl.ds(wid * _EPT, _EPT)], dstv)
    pltpu.sync_copy(pos_hbm.at[pl.ds(wid * _EPT, _EPT)], posv)
    pltpu.sync_copy(fill_hbm.at[pl.ds(wid * _NB * _CK, _NB * _CK)], fidxb)

    def chunk(j, _):
        slot = lax.rem(j, 4)

        @pl.when(j >= 4)
        def _():
            pltpu.make_async_copy(datb.at[0], bins_hbm.at[idxb.at[0]],
                                  sem).wait()

        for u in range(_CK // 16):
            o = j * _CK + u * 16
            s16 = srcv[pl.ds(o, 16)]
            d16 = dstv[pl.ds(o, 16)]
            b16 = (d16 * 18894) >> 25
            idxb[slot, pl.ds(u * 16, 16)] = posv[pl.ds(o, 16)]
            datb[slot, pl.ds(u * 16, 16)] = (d16 - b16 * _RNG) * 16384 + s16
        pltpu.async_copy(datb.at[slot], bins_hbm.at[idxb.at[slot]], sem)
        return 0

    nfull = _EPT // _CK
    lax.fori_loop(0, nfull, chunk, 0)
    for _u in range(4):
        pltpu.make_async_copy(datb.at[0], bins_hbm.at[idxb.at[0]],
                              sem).wait()
    o = nfull * _CK
    ntail = _EPT - o
    for u in range(ntail // 16):
        s16 = srcv[pl.ds(o + u * 16, 16)]
        d16 = dstv[pl.ds(o + u * 16, 16)]
        b16 = (d16 * 18894) >> 25
        idxb[0, pl.ds(u * 16, 16)] = posv[pl.ds(o + u * 16, 16)]
        datb[0, pl.ds(u * 16, 16)] = (d16 - b16 * _RNG) * 16384 + s16
    pltpu.sync_copy(datb.at[0, pl.ds(0, 16)],
                    bins_hbm.at[idxb.at[0, pl.ds(0, 16)]])

    dump16 = jnp.full((16,), _PACKDUMP, jnp.int32)
    for u in range(_CK // 16):
        datb[0, pl.ds(u * 16, 16)] = dump16
    for j in range(_NB):
        slot = 1 + (j % 3)
        if j >= 3:
            pltpu.make_async_copy(datb.at[0], bins_hbm.at[idxb.at[0]],
                                  sem).wait()
        for u in range(_CK // 16):
            idxb[slot, pl.ds(u * 16, 16)] = fidxb[pl.ds(j * _CK + u * 16, 16)]
        pltpu.async_copy(datb.at[0], bins_hbm.at[idxb.at[slot]], sem)
    for _u in range(3):
        pltpu.make_async_copy(datb.at[0], bins_hbm.at[idxb.at[0]],
                              sem).wait()


def _agg_body(bins_hbm, cnts_hbm, g_hbm, out_hbm, pk, srcb, dstb, rows, zb,
              wb, cbufa, cbufb, agg_sh, bsem, gsem, ssem):
    c = lax.axis_index("c")
    s = lax.axis_index("s")

    z = jnp.zeros((16,), jnp.float32)

    def zz(i, _):
        for t in range(_DP // 16):
            zb[i, pl.ds(t * 16, 16)] = z
        return 0

    lax.fori_loop(0, _ARPT, zz, 0)

    pltpu.sync_copy(cnts_hbm.at[pl.ds((2 * s) * 16, 16)], cbufa)
    pltpu.sync_copy(cnts_hbm.at[pl.ds((2 * s + 1) * 16, 16)], cbufb)


    cva = cbufa[...]
    cvb = cbufb[...]

    for r in range(_NR):
        b = 2 * r + c
        pltpu.sync_copy(zb, agg_sh.at[pl.ds(s * _ARPT, _ARPT)])
        plsc.subcore_barrier()

        ma = jnp.where(c == 0, cva[2 * r], cva[2 * r + 1])
        mb = jnp.where(c == 0, cvb[2 * r], cvb[2 * r + 1])
        mt = ma + mb
        base_a = ((2 * s) * _NB + b) * _CAP
        base_b = ((2 * s + 1) * _NB + b) * _CAP

        def off(j):
            return jnp.where(j < ma, base_a + j * _CK,
                             base_b + (j - ma) * _CK)

        def bcopy(j, slot):
            pltpu.async_copy(bins_hbm.at[pl.ds(off(j), _CK)],
                             pk.at[slot], bsem)

        for k in range(3):
            @pl.when(k < mt)
            def _():
                bcopy(k, k)

        def chunk(j, _):
            slot = lax.rem(j, 4)

            @pl.when(j >= 4)
            def _():
                pltpu.make_async_copy(
                    rows.at[0], agg_sh.at[dstb.at[0]], ssem).wait()

            pltpu.make_async_copy(
                bins_hbm.at[pl.ds(0, _CK)], pk.at[0], bsem).wait()
            for u in range(_CK // 16):
                p = pk[slot, pl.ds(u * 16, 16)]
                srcb[slot, pl.ds(u * 16, 16)] = p & 16383
                dstb[slot, pl.ds(u * 16, 16)] = p >> 14

            @pl.when(j + 3 < mt)
            def _():
                bcopy(j + 3, lax.rem(j + 3, 4))

            pltpu.async_copy(g_hbm.at[srcb.at[slot]], rows.at[slot], gsem)

            @pl.when(j >= 1)
            def _():
                prev = lax.rem(j - 1, 4)
                pltpu.make_async_copy(
                    g_hbm.at[srcb.at[0]], rows.at[0], gsem).wait()
                pltpu.async_copy(rows.at[prev], agg_sh.at[dstb.at[prev]],
                                 ssem, add=True)
            return 0

        lax.fori_loop(0, mt, chunk, 0)

        @pl.when(mt >= 1)
        def _():
            last = lax.rem(mt - 1, 4)
            pltpu.make_async_copy(
                g_hbm.at[srcb.at[0]], rows.at[0], gsem).wait()
            pltpu.async_copy(rows.at[last], agg_sh.at[dstb.at[last]],
                             ssem, add=True)

        def drain(j, _):
            pltpu.make_async_copy(
                rows.at[0], agg_sh.at[dstb.at[0]], ssem).wait()
            return 0

        lax.fori_loop(0, jnp.minimum(mt, 4), drain, 0)

        plsc.subcore_barrier()
        pltpu.sync_copy(agg_sh.at[pl.ds(s * _ARPT, _ARPT)], wb)
        pltpu.sync_copy(wb, out_hbm.at[pl.ds(b * _ACC + s * _ARPT, _ARPT)])
        plsc.subcore_barrier()


@functools.cache
def _sc_kernels():
    mesh = plsc.VectorSubcoreMesh(
        core_axis_name="c", subcore_axis_name="s",
        num_cores=_NC, num_subcores=_NS,
    )
    deg = functools.partial(
        pl.kernel,
        out_type=jax.ShapeDtypeStruct((_NC * _NP,), jnp.float32),
        mesh=mesh,
        scratch_types=[
            pltpu.VMEM((_NCH, _K), jnp.int32),
            pltpu.VMEM((_K,), jnp.float32),
            pltpu.VMEM((_DRPT,), jnp.float32),
            pltpu.VMEM_SHARED((_NP,), jnp.float32),
            pltpu.SemaphoreType.DMA,
        ],
    )(_deg_body)
    binf = functools.partial(
        pl.kernel,
        out_type=jax.ShapeDtypeStruct((_NW * _TS + _CK,), jnp.int32),
        mesh=mesh,
        scratch_types=[
            pltpu.VMEM((_EPT,), jnp.int32),
            pltpu.VMEM((_EPT,), jnp.int32),
            pltpu.VMEM((_EPT,), jnp.int32),
            pltpu.VMEM((4, _CK), jnp.int32),
            pltpu.VMEM((4, _CK), jnp.int32),
            pltpu.VMEM((_NB * _CK,), jnp.int32),
            pltpu.SemaphoreType.DMA,
        ],
    )(_bin_body)
    agg = functools.partial(
        pl.kernel,
        out_type=jax.ShapeDtypeStruct((_OUTR, _DP), jnp.float32),
        mesh=mesh,
        scratch_types=[
            pltpu.VMEM((4, _CK), jnp.int32),
            pltpu.VMEM((4, _CK), jnp.int32),
            pltpu.VMEM((4, _CK), jnp.int32),
            pltpu.VMEM((4, _CK, _DP), jnp.float32),
            pltpu.VMEM((_ARPT, _DP), jnp.float32),
            pltpu.VMEM((_ARPT, _DP), jnp.float32),
            pltpu.VMEM((16,), jnp.int32),
            pltpu.VMEM((16,), jnp.int32),
            pltpu.VMEM_SHARED((_ACC, _DP), jnp.float32),
            pltpu.SemaphoreType.DMA,
            pltpu.SemaphoreType.DMA,
            pltpu.SemaphoreType.DMA,
        ],
    )(_agg_body)
    return deg, binf, agg


def _gcn_in(aggp_ref):
    parts = []
    done = 0
    for b in range(_NB):
        n = min(_RNG, _N - done)
        if n > 0:
            parts.append(aggp_ref[b * _ACC:b * _ACC + n, :_DH])
        done += n
    return jnp.concatenate(parts, axis=0)


def _pos_body(dst3_ref, pos_ref, fill_ref, cnts_ref):
    d = dst3_ref[...]
    bkt = (d * 18894) >> 25
    jrow = lax.broadcasted_iota(jnp.int32, (625, 625), 0)
    icol = lax.broadcasted_iota(jnp.int32, (625, 625), 1)
    triv = (jrow < icol).astype(jnp.float32)
    j16 = lax.broadcasted_iota(jnp.int32, (16, 16), 0)
    i16 = lax.broadcasted_iota(jnp.int32, (16, 16), 1)
    tril = (j16 < i16).astype(jnp.float32)
    t = pl.program_id(0)
    bidx16 = lax.broadcasted_iota(jnp.int32, (1, 1, 16), 2)
    kfid = lax.broadcasted_iota(jnp.int32, (1, _CK), 1)
    pos = jnp.zeros((1, 625, 16), jnp.int32)
    cnts = jnp.zeros((1, 1, 16), jnp.int32)
    fills = []
    for b in range(_NB):
        oh = (bkt == b).astype(jnp.float32)
        vcnt = jnp.sum(oh, axis=2)
        vpre = lax.dot_general(vcnt, triv, (((1,), (0,)), ((), ())),
                               preferred_element_type=jnp.float32)
        lpre = lax.dot_general(oh, tril, (((2,), (0,)), ((), ())),
                               preferred_element_type=jnp.float32)
        rank = (vpre[:, :, None] + lpre).astype(jnp.int32)
        pos = pos + jnp.where(bkt == b, b * _CAP + rank, 0)
        tot = (vpre[:, 624] + vcnt[:, 624]).astype(jnp.int32)
        ncb = (tot + _CK - 1) // _CK
        cnts = jnp.where(bidx16 == b, ncb[:, None, None], cnts)
        nfill = ncb * _CK - tot
        fp = jnp.where(kfid < nfill[:, None],
                       b * _CAP + tot[:, None] + kfid + t * _TS,
                       _TRASH)
        fills.append(fp[:, None, :])
    pos_ref[...] = pos + t * _TS
    fill_ref[...] = jnp.concatenate(fills, axis=1)
    cnts_ref[...] = cnts


_pos_call = pl.pallas_call(
    _pos_body,
    grid=(_NW,),
    in_specs=[pl.BlockSpec((1, 625, 16), lambda t: (t, 0, 0))],
    out_specs=[
        pl.BlockSpec((1, 625, 16), lambda t: (t, 0, 0)),
        pl.BlockSpec((1, _NB, _CK), lambda t: (t, 0, 0)),
        pl.BlockSpec((1, 1, 16), lambda t: (t, 0, 0)),
    ],
    out_shape=[
        jax.ShapeDtypeStruct((_NW, 625, 16), jnp.int32),
        jax.ShapeDtypeStruct((_NW, _NB, _CK), jnp.int32),
        jax.ShapeDtypeStruct((_NW, 1, 16), jnp.int32),
    ],
)


def _prep_body(degp_ref, x_ref, w1_ref, g1_ref, dinv_ref):
    deg = degp_ref[0, :_N] + degp_ref[1, :_N] + 1.0
    dinv = lax.rsqrt(deg).reshape(_N, 1)
    h = jnp.dot(x_ref[...], w1_ref[...], preferred_element_type=jnp.float32)
    g1_ref[...] = h * dinv
    dinv_ref[...] = dinv


def _mid_body(aggp_ref, g_ref, dinv_ref, b1_ref, w2_ref, g2_ref):
    agg = _gcn_in(aggp_ref) + g_ref[:, :_DH]
    h = jnp.maximum(agg * dinv_ref[...] + b1_ref[...], 0.0)
    g2_ref[...] = jnp.dot(
        h, w2_ref[...], preferred_element_type=jnp.float32) * dinv_ref[...]


def _fin_body(aggp_ref, g2_ref, dinv_ref, b2_ref, batch_ref, wfc_ref, bfc_ref,
              out_ref):
    agg = _gcn_in(aggp_ref) + g2_ref[:, :_DH]
    h = jnp.maximum(agg * dinv_ref[...] + b2_ref[...], 0.0)
    onehot = (batch_ref[...] == lax.broadcasted_iota(
        jnp.int32, (1, _G), 1)).astype(jnp.float32)
    pooled = lax.dot_general(onehot, h, (((0,), (0,)), ((), ())),
                             preferred_element_type=jnp.float32)
    cnt = lax.dot_general(onehot, jnp.ones((_N, 1), jnp.float32),
                          (((0,), (0,)), ((), ())),
                          preferred_element_type=jnp.float32)
    pooled = pooled / jnp.maximum(cnt, 1.0)
    out_ref[...] = jnp.dot(
        pooled, wfc_ref[...], preferred_element_type=jnp.float32) + bfc_ref[...]


_prep_call = pl.pallas_call(
    _prep_body,
    out_shape=[
        jax.ShapeDtypeStruct((_N, _DP), jnp.float32),
        jax.ShapeDtypeStruct((_N, 1), jnp.float32),
    ],
)

_mid_call = pl.pallas_call(
    _mid_body,
    out_shape=jax.ShapeDtypeStruct((_N, _DP), jnp.float32),
)

_fin_call = pl.pallas_call(
    _fin_body,
    out_shape=jax.ShapeDtypeStruct((_G, 1), jnp.float32),
)


def kernel(x, edge_index, batch, W1, b1, W2, b2, Wfc, bfc):
    _deg_kernel, _bin_kernel, _agg_kernel = _sc_kernels()
    src1d = edge_index[0].reshape(_E)
    dst1d = edge_index[1].reshape(_E)
    dst_deg = edge_index[1].reshape(_NW, _NCH, _K)
    w1p = jnp.pad(W1, ((0, 0), (0, _DP - _DH)))
    w2p = jnp.pad(W2, ((0, 0), (0, _DP - _DH)))

    dst3 = edge_index[1].reshape(_NW, 625, 16)
    post, fill3, cnts2d = _pos_call(dst3)
    pos1d = post.reshape(_E)
    fill1d = fill3.reshape(_NW * _NB * _CK)
    cnts = cnts2d.reshape(_NW * 16)

    degp = _deg_kernel(dst_deg).reshape(_NC, _NP)
    bins = _bin_kernel(src1d, dst1d, pos1d, fill1d)
    g1, dinv = _prep_call(degp, x, w1p)
    aggp1 = _agg_kernel(bins, cnts, g1)
    g2 = _mid_call(aggp1, g1, dinv, b1.reshape(1, _DH), w2p)
    aggp2 = _agg_kernel(bins, cnts, g2)
    out = _fin_call(aggp2, g2, dinv, b2.reshape(1, _DH), batch.reshape(_N, 1),
                    Wfc, bfc.reshape(1, 1))
    return out


# speedup vs baseline: 6.2234x; 2.2151x over previous
import functools

import jax
import jax.numpy as jnp
from jax import lax
from jax.experimental import pallas as pl
from jax.experimental.pallas import tpu as pltpu
from jax.experimental.pallas import tpu_sc as plsc

_N = 10000
_E = 320000
_DIN = 128
_DH = 64
_G = 64

_NC = 2
_NS = 16
_NW = _NC * _NS
_DP = 128

_K = 125
_EPT = _E // _NW
_NCH = _EPT // _K
_DRPT = 640
_NP = _DRPT * _NS

_NB = 6
_RNG = 1776
_CAP = 10240
_CK = 128
_DUMP = 1791
_PACKDUMP = _DUMP * 16384

_TS = _NB * _CAP
_TRASHL = _NS * _CAP
_ACC = 1792
_ARPT = _ACC // _NS
_NR = 3
_OUTR = _NB * _ACC


def _deg_body(dst_hbm, out_hbm, dstv, onesv, wb, deg_sh, sem):
    c = lax.axis_index("c")
    s = lax.axis_index("s")
    wid = s * _NC + c

    pltpu.sync_copy(dst_hbm.at[wid], dstv)

    one = jnp.ones((16,), jnp.float32)
    for i in range(_K // 16):
        onesv[pl.ds(i * 16, 16)] = one
    onesv[pl.ds(_K - 16, 16)] = one

    z = jnp.zeros((16,), jnp.float32)

    def zb(i, _):
        wb[pl.ds(i * 16, 16)] = z
        return 0

    lax.fori_loop(0, _DRPT // 16, zb, 0)
    pltpu.sync_copy(wb, deg_sh.at[pl.ds(s * _DRPT, _DRPT)])
    plsc.subcore_barrier()

    def body(j, _):
        pltpu.async_copy(onesv, deg_sh.at[dstv.at[j]], sem, add=True)
        return 0

    lax.fori_loop(0, _NCH, body, 0)

    def draindeg(j, _):
        pltpu.make_async_copy(onesv, deg_sh.at[dstv.at[0]], sem).wait()
        return 0

    lax.fori_loop(0, _NCH, draindeg, 0)
    plsc.subcore_barrier()

    pltpu.sync_copy(deg_sh.at[pl.ds(s * _DRPT, _DRPT)], wb)
    pltpu.sync_copy(wb, out_hbm.at[pl.ds(c * _NP + s * _DRPT, _DRPT)])


def _bin_body(src_hbm, dst_hbm, pos_hbm, fill_hbm, bins_hbm, srcv, dstv,
              posv, idxb, datb, fidxb, wbuf, region, sem):
    c = lax.axis_index("c")
    s = lax.axis_index("s")
    wid = s * _NC + c

    pltpu.sync_copy(src_hbm.at[pl.ds(wid * _EPT, _EPT)], srcv)
    pltpu.sync_copy(dst_hbm.at[pl.ds(wid * _EPT, _EPT)], dstv)
    pltpu.sync_copy(pos_hbm.at[pl.ds(wid * _EPT, _EPT)], posv)
    pltpu.sync_copy(fill_hbm.at[pl.ds(wid * _NB * _CK, _NB * _CK)], fidxb)

    dump16 = jnp.full((16,), _PACKDUMP, jnp.int32)
    nfull = _EPT // _CK
    o = nfull * _CK
    ntail = _EPT - o

    for b in range(_NB):

        def chunk(j, _):
            slot = lax.rem(j, 4)

            @pl.when(j >= 4)
            def _():
                pltpu.make_async_copy(datb.at[0], region.at[idxb.at[0]],
                                      sem).wait()

            for u in range(_CK // 16):
                oo = j * _CK + u * 16
                s16 = srcv[pl.ds(oo, 16)]
                d16 = dstv[pl.ds(oo, 16)]
                p16 = posv[pl.ds(oo, 16)]
                b16 = (d16 * 18894) >> 25
                idxb[slot, pl.ds(u * 16, 16)] = jnp.where(
                    b16 == b, s * _CAP + (p16 - b * _CAP), _TRASHL)
                datb[slot, pl.ds(u * 16, 16)] = (
                    (d16 - b16 * _RNG) * 16384 + s16)
            pltpu.async_copy(datb.at[slot], region.at[idxb.at[slot]], sem)
            return 0

        lax.fori_loop(0, nfull, chunk, 0)
        for _u in range(4):
            pltpu.make_async_copy(datb.at[0], region.at[idxb.at[0]],
                                  sem).wait()
        for u in range(ntail // 16):
            s16 = srcv[pl.ds(o + u * 16, 16)]
            d16 = dstv[pl.ds(o + u * 16, 16)]
            p16 = posv[pl.ds(o + u * 16, 16)]
            b16 = (d16 * 18894) >> 25
            idxb[0, pl.ds(u * 16, 16)] = jnp.where(
                b16 == b, s * _CAP + (p16 - b * _CAP), _TRASHL)
            datb[0, pl.ds(u * 16, 16)] = (d16 - b16 * _RNG) * 16384 + s16
        pltpu.sync_copy(datb.at[0, pl.ds(0, 16)],
                        region.at[idxb.at[0, pl.ds(0, 16)]])

        for u in range(_CK // 16):
            fv = fidxb[pl.ds(b * _CK + u * 16, 16)]
            idxb[0, pl.ds(u * 16, 16)] = jnp.where(
                fv != _TS, s * _CAP + (fv - b * _CAP), _TRASHL)
            datb[0, pl.ds(u * 16, 16)] = dump16
        pltpu.sync_copy(datb.at[0], region.at[idxb.at[0]])

        plsc.subcore_barrier()
        pltpu.sync_copy(region.at[pl.ds(s * _CAP, _CAP)], wbuf)
        pltpu.sync_copy(wbuf,
                        bins_hbm.at[pl.ds(wid * _TS + b * _CAP, _CAP)])
        plsc.subcore_barrier()


def _agg_body(bins_hbm, cnts_hbm, g_hbm, out_hbm, pk, srcb, dstb, rows, zb,
              wb, cbufa, cbufb, agg_sh, bsem, gsem, ssem):
    c = lax.axis_index("c")
    s = lax.axis_index("s")

    z = jnp.zeros((16,), jnp.float32)

    def zz(i, _):
        for t in range(_DP // 16):
            zb[i, pl.ds(t * 16, 16)] = z
        return 0

    lax.fori_loop(0, _ARPT, zz, 0)

    pltpu.sync_copy(cnts_hbm.at[pl.ds((2 * s) * 16, 16)], cbufa)
    pltpu.sync_copy(cnts_hbm.at[pl.ds((2 * s + 1) * 16, 16)], cbufb)


    cva = cbufa[...]
    cvb = cbufb[...]

    for r in range(_NR):
        b = 2 * r + c
        pltpu.sync_copy(zb, agg_sh.at[pl.ds(s * _ARPT, _ARPT)])
        plsc.subcore_barrier()

        ma = jnp.where(c == 0, cva[2 * r], cva[2 * r + 1])
        mb = jnp.where(c == 0, cvb[2 * r], cvb[2 * r + 1])
        mt = ma + mb
        base_a = ((2 * s) * _NB + b) * _CAP
        base_b = ((2 * s + 1) * _NB + b) * _CAP

        def off(j):
            return jnp.where(j < ma, base_a + j * _CK,
                             base_b + (j - ma) * _CK)

        def bcopy(j, slot):
            pltpu.async_copy(bins_hbm.at[pl.ds(off(j), _CK)],
                             pk.at[slot], bsem)

        for k in range(3):
            @pl.when(k < mt)
            def _():
                bcopy(k, k)

        def chunk(j, _):
            slot = lax.rem(j, 4)

            @pl.when(j >= 4)
            def _():
                pltpu.make_async_copy(
                    rows.at[0], agg_sh.at[dstb.at[0]], ssem).wait()

            pltpu.make_async_copy(
                bins_hbm.at[pl.ds(0, _CK)], pk.at[0], bsem).wait()
            for u in range(_CK // 16):
                p = pk[slot, pl.ds(u * 16, 16)]
                srcb[slot, pl.ds(u * 16, 16)] = p & 16383
                dstb[slot, pl.ds(u * 16, 16)] = p >> 14

            @pl.when(j + 3 < mt)
            def _():
                bcopy(j + 3, lax.rem(j + 3, 4))

            pltpu.async_copy(g_hbm.at[srcb.at[slot]], rows.at[slot], gsem)

            @pl.when(j >= 1)
            def _():
                prev = lax.rem(j - 1, 4)
                pltpu.make_async_copy(
                    g_hbm.at[srcb.at[0]], rows.at[0], gsem).wait()
                pltpu.async_copy(rows.at[prev], agg_sh.at[dstb.at[prev]],
                                 ssem, add=True)
            return 0

        lax.fori_loop(0, mt, chunk, 0)

        @pl.when(mt >= 1)
        def _():
            last = lax.rem(mt - 1, 4)
            pltpu.make_async_copy(
                g_hbm.at[srcb.at[0]], rows.at[0], gsem).wait()
            pltpu.async_copy(rows.at[last], agg_sh.at[dstb.at[last]],
                             ssem, add=True)

        def drain(j, _):
            pltpu.make_async_copy(
                rows.at[0], agg_sh.at[dstb.at[0]], ssem).wait()
            return 0

        lax.fori_loop(0, jnp.minimum(mt, 4), drain, 0)

        plsc.subcore_barrier()
        pltpu.sync_copy(agg_sh.at[pl.ds(s * _ARPT, _ARPT)], wb)
        pltpu.sync_copy(wb, out_hbm.at[pl.ds(b * _ACC + s * _ARPT, _ARPT)])
        plsc.subcore_barrier()


@functools.cache
def _sc_kernels():
    mesh = plsc.VectorSubcoreMesh(
        core_axis_name="c", subcore_axis_name="s",
        num_cores=_NC, num_subcores=_NS,
    )
    deg = functools.partial(
        pl.kernel,
        out_type=jax.ShapeDtypeStruct((_NC * _NP,), jnp.float32),
        mesh=mesh,
        scratch_types=[
            pltpu.VMEM((_NCH, _K), jnp.int32),
            pltpu.VMEM((_K,), jnp.float32),
            pltpu.VMEM((_DRPT,), jnp.float32),
            pltpu.VMEM_SHARED((_NP,), jnp.float32),
            pltpu.SemaphoreType.DMA,
        ],
    )(_deg_body)
    binf = functools.partial(
        pl.kernel,
        out_type=jax.ShapeDtypeStruct((_NW * _TS,), jnp.int32),
        mesh=mesh,
        scratch_types=[
            pltpu.VMEM((_EPT,), jnp.int32),
            pltpu.VMEM((_EPT,), jnp.int32),
            pltpu.VMEM((_EPT,), jnp.int32),
            pltpu.VMEM((4, _CK), jnp.int32),
            pltpu.VMEM((4, _CK), jnp.int32),
            pltpu.VMEM((_NB * _CK,), jnp.int32),
            pltpu.VMEM((_CAP,), jnp.int32),
            pltpu.VMEM_SHARED((_NS * _CAP + 16,), jnp.int32),
            pltpu.SemaphoreType.DMA,
        ],
    )(_bin_body)
    agg = functools.partial(
        pl.kernel,
        out_type=jax.ShapeDtypeStruct((_OUTR, _DP), jnp.float32),
        mesh=mesh,
        scratch_types=[
            pltpu.VMEM((4, _CK), jnp.int32),
            pltpu.VMEM((4, _CK), jnp.int32),
            pltpu.VMEM((4, _CK), jnp.int32),
            pltpu.VMEM((4, _CK, _DP), jnp.float32),
            pltpu.VMEM((_ARPT, _DP), jnp.float32),
            pltpu.VMEM((_ARPT, _DP), jnp.float32),
            pltpu.VMEM((16,), jnp.int32),
            pltpu.VMEM((16,), jnp.int32),
            pltpu.VMEM_SHARED((_ACC, _DP), jnp.float32),
            pltpu.SemaphoreType.DMA,
            pltpu.SemaphoreType.DMA,
            pltpu.SemaphoreType.DMA,
        ],
    )(_agg_body)
    return deg, binf, agg


def _gcn_in(aggp_ref):
    parts = []
    done = 0
    for b in range(_NB):
        n = min(_RNG, _N - done)
        if n > 0:
            parts.append(aggp_ref[b * _ACC:b * _ACC + n, :_DH])
        done += n
    return jnp.concatenate(parts, axis=0)


def _pos_body(dst3_ref, pos_ref, fill_ref, cnts_ref):
    d = dst3_ref[...]
    bkt = (d * 18894) >> 25
    jrow = lax.broadcasted_iota(jnp.int32, (625, 625), 0)
    icol = lax.broadcasted_iota(jnp.int32, (625, 625), 1)
    triv = (jrow < icol).astype(jnp.float32)
    j16 = lax.broadcasted_iota(jnp.int32, (16, 16), 0)
    i16 = lax.broadcasted_iota(jnp.int32, (16, 16), 1)
    tril = (j16 < i16).astype(jnp.float32)
    t = pl.program_id(0)
    bidx16 = lax.broadcasted_iota(jnp.int32, (1, 1, 16), 2)
    kfid = lax.broadcasted_iota(jnp.int32, (1, _CK), 1)
    pos = jnp.zeros((1, 625, 16), jnp.int32)
    cnts = jnp.zeros((1, 1, 16), jnp.int32)
    fills = []
    for b in range(_NB):
        oh = (bkt == b).astype(jnp.float32)
        vcnt = jnp.sum(oh, axis=2)
        vpre = lax.dot_general(vcnt, triv, (((1,), (0,)), ((), ())),
                               preferred_element_type=jnp.float32)
        lpre = lax.dot_general(oh, tril, (((2,), (0,)), ((), ())),
                               preferred_element_type=jnp.float32)
        rank = (vpre[:, :, None] + lpre).astype(jnp.int32)
        pos = pos + jnp.where(bkt == b, b * _CAP + rank, 0)
        tot = (vpre[:, 624] + vcnt[:, 624]).astype(jnp.int32)
        ncb = (tot + _CK - 1) // _CK
        cnts = jnp.where(bidx16 == b, ncb[:, None, None], cnts)
        nfill = ncb * _CK - tot
        fp = jnp.where(kfid < nfill[:, None],
                       b * _CAP + tot[:, None] + kfid,
                       _TS)
        fills.append(fp[:, None, :])
    pos_ref[...] = pos + t * 0
    fill_ref[...] = jnp.concatenate(fills, axis=1)
    cnts_ref[...] = cnts


_pos_call = pl.pallas_call(
    _pos_body,
    grid=(_NW,),
    in_specs=[pl.BlockSpec((1, 625, 16), lambda t: (t, 0, 0))],
    out_specs=[
        pl.BlockSpec((1, 625, 16), lambda t: (t, 0, 0)),
        pl.BlockSpec((1, _NB, _CK), lambda t: (t, 0, 0)),
        pl.BlockSpec((1, 1, 16), lambda t: (t, 0, 0)),
    ],
    out_shape=[
        jax.ShapeDtypeStruct((_NW, 625, 16), jnp.int32),
        jax.ShapeDtypeStruct((_NW, _NB, _CK), jnp.int32),
        jax.ShapeDtypeStruct((_NW, 1, 16), jnp.int32),
    ],
)


def _prep_body(degp_ref, x_ref, w1_ref, g1_ref, dinv_ref):
    deg = degp_ref[0, :_N] + degp_ref[1, :_N] + 1.0
    dinv = lax.rsqrt(deg).reshape(_N, 1)
    h = jnp.dot(x_ref[...], w1_ref[...], preferred_element_type=jnp.float32)
    g1_ref[...] = h * dinv
    dinv_ref[...] = dinv


def _mid_body(aggp_ref, g_ref, dinv_ref, b1_ref, w2_ref, g2_ref):
    agg = _gcn_in(aggp_ref) + g_ref[:, :_DH]
    h = jnp.maximum(agg * dinv_ref[...] + b1_ref[...], 0.0)
    g2_ref[...] = jnp.dot(
        h, w2_ref[...], preferred_element_type=jnp.float32) * dinv_ref[...]


def _fin_body(aggp_ref, g2_ref, dinv_ref, b2_ref, batch_ref, wfc_ref, bfc_ref,
              out_ref):
    agg = _gcn_in(aggp_ref) + g2_ref[:, :_DH]
    h = jnp.maximum(agg * dinv_ref[...] + b2_ref[...], 0.0)
    onehot = (batch_ref[...] == lax.broadcasted_iota(
        jnp.int32, (1, _G), 1)).astype(jnp.float32)
    pooled = lax.dot_general(onehot, h, (((0,), (0,)), ((), ())),
                             preferred_element_type=jnp.float32)
    cnt = lax.dot_general(onehot, jnp.ones((_N, 1), jnp.float32),
                          (((0,), (0,)), ((), ())),
                          preferred_element_type=jnp.float32)
    pooled = pooled / jnp.maximum(cnt, 1.0)
    out_ref[...] = jnp.dot(
        pooled, wfc_ref[...], preferred_element_type=jnp.float32) + bfc_ref[...]


_prep_call = pl.pallas_call(
    _prep_body,
    out_shape=[
        jax.ShapeDtypeStruct((_N, _DP), jnp.float32),
        jax.ShapeDtypeStruct((_N, 1), jnp.float32),
    ],
)

_mid_call = pl.pallas_call(
    _mid_body,
    out_shape=jax.ShapeDtypeStruct((_N, _DP), jnp.float32),
)

_fin_call = pl.pallas_call(
    _fin_body,
    out_shape=jax.ShapeDtypeStruct((_G, 1), jnp.float32),
)


def kernel(x, edge_index, batch, W1, b1, W2, b2, Wfc, bfc):
    _deg_kernel, _bin_kernel, _agg_kernel = _sc_kernels()
    src1d = edge_index[0].reshape(_E)
    dst1d = edge_index[1].reshape(_E)
    dst_deg = edge_index[1].reshape(_NW, _NCH, _K)
    w1p = jnp.pad(W1, ((0, 0), (0, _DP - _DH)))
    w2p = jnp.pad(W2, ((0, 0), (0, _DP - _DH)))

    dst3 = edge_index[1].reshape(_NW, 625, 16)
    post, fill3, cnts2d = _pos_call(dst3)
    pos1d = post.reshape(_E)
    fill1d = fill3.reshape(_NW * _NB * _CK)
    cnts = cnts2d.reshape(_NW * 16)

    degp = _deg_kernel(dst_deg).reshape(_NC, _NP)
    bins = _bin_kernel(src1d, dst1d, pos1d, fill1d)
    g1, dinv = _prep_call(degp, x, w1p)
    aggp1 = _agg_kernel(bins, cnts, g1)
    g2 = _mid_call(aggp1, g1, dinv, b1.reshape(1, _DH), w2p)
    aggp2 = _agg_kernel(bins, cnts, g2)
    out = _fin_call(aggp2, g2, dinv, b2.reshape(1, _DH), batch.reshape(_N, 1),
                    Wfc, bfc.reshape(1, 1))
    return out


# speedup vs baseline: 8.4523x; 1.3581x over previous
import functools

import jax
import jax.numpy as jnp
from jax import lax
from jax.experimental import pallas as pl
from jax.experimental.pallas import tpu as pltpu
from jax.experimental.pallas import tpu_sc as plsc

_N = 10000
_E = 320000
_DIN = 128
_DH = 64
_G = 64

_NC = 2
_NS = 16
_NW = _NC * _NS
_DP = 128

_K = 125
_EPT = _E // _NW
_NCH = _EPT // _K
_DRPT = 640
_NP = _DRPT * _NS

_NB = 6
_RNG = 1776
_CAP = 10240
_CK = 128
_DUMP = 1791
_PACKDUMP = _DUMP * 16384

_TS = _NB * _CAP
_GRP = 4
_TRASHL = _GRP * _NB * _CAP
_ACC = 1792
_ARPT = _ACC // _NS
_NR = 3
_OUTR = _NB * _ACC


def _deg_body(dst_hbm, out_hbm, dstv, onesv, wb, deg_sh, sem):
    c = lax.axis_index("c")
    s = lax.axis_index("s")
    wid = s * _NC + c

    pltpu.sync_copy(dst_hbm.at[wid], dstv)

    one = jnp.ones((16,), jnp.float32)
    for i in range(_K // 16):
        onesv[pl.ds(i * 16, 16)] = one
    onesv[pl.ds(_K - 16, 16)] = one

    z = jnp.zeros((16,), jnp.float32)

    def zb(i, _):
        wb[pl.ds(i * 16, 16)] = z
        return 0

    lax.fori_loop(0, _DRPT // 16, zb, 0)
    pltpu.sync_copy(wb, deg_sh.at[pl.ds(s * _DRPT, _DRPT)])
    plsc.subcore_barrier()

    def body(j, _):
        pltpu.async_copy(onesv, deg_sh.at[dstv.at[j]], sem, add=True)
        return 0

    lax.fori_loop(0, _NCH, body, 0)

    def draindeg(j, _):
        pltpu.make_async_copy(onesv, deg_sh.at[dstv.at[0]], sem).wait()
        return 0

    lax.fori_loop(0, _NCH, draindeg, 0)
    plsc.subcore_barrier()

    pltpu.sync_copy(deg_sh.at[pl.ds(s * _DRPT, _DRPT)], wb)
    pltpu.sync_copy(wb, out_hbm.at[pl.ds(c * _NP + s * _DRPT, _DRPT)])


def _bin_body(src_hbm, dst_hbm, pos_hbm, fill_hbm, bins_hbm, srcv, dstv,
              posv, idxb, datb, fidxb, wbuf, region, sem):
    c = lax.axis_index("c")
    s = lax.axis_index("s")
    wid = s * _NC + c

    pltpu.sync_copy(src_hbm.at[pl.ds(wid * _EPT, _EPT)], srcv)
    pltpu.sync_copy(dst_hbm.at[pl.ds(wid * _EPT, _EPT)], dstv)
    pltpu.sync_copy(pos_hbm.at[pl.ds(wid * _EPT, _EPT)], posv)
    pltpu.sync_copy(fill_hbm.at[pl.ds(wid * _NB * _CK, _NB * _CK)], fidxb)

    dump16 = jnp.full((16,), _PACKDUMP, jnp.int32)
    nfull = _EPT // _CK
    o = nfull * _CK
    ntail = _EPT - o
    base = lax.rem(s, 4) * _TS
    gslice = _GRP * _TS // _NS

    def scatter_all():
        def chunk(j, _):
            slot = lax.rem(j, 4)

            @pl.when(j >= 4)
            def _():
                pltpu.make_async_copy(datb.at[0], region.at[idxb.at[0]],
                                      sem).wait()

            for u in range(_CK // 16):
                oo = j * _CK + u * 16
                s16 = srcv[pl.ds(oo, 16)]
                d16 = dstv[pl.ds(oo, 16)]
                b16 = (d16 * 18894) >> 25
                idxb[slot, pl.ds(u * 16, 16)] = base + posv[pl.ds(oo, 16)]
                datb[slot, pl.ds(u * 16, 16)] = (
                    (d16 - b16 * _RNG) * 16384 + s16)
            pltpu.async_copy(datb.at[slot], region.at[idxb.at[slot]], sem)
            return 0

        lax.fori_loop(0, nfull, chunk, 0)
        for _u in range(4):
            pltpu.make_async_copy(datb.at[0], region.at[idxb.at[0]],
                                  sem).wait()
        for u in range(ntail // 16):
            s16 = srcv[pl.ds(o + u * 16, 16)]
            d16 = dstv[pl.ds(o + u * 16, 16)]
            b16 = (d16 * 18894) >> 25
            idxb[0, pl.ds(u * 16, 16)] = base + posv[pl.ds(o + u * 16, 16)]
            datb[0, pl.ds(u * 16, 16)] = (d16 - b16 * _RNG) * 16384 + s16
        pltpu.sync_copy(datb.at[0, pl.ds(0, 16)],
                        region.at[idxb.at[0, pl.ds(0, 16)]])
        for u in range(_CK // 16):
            datb[0, pl.ds(u * 16, 16)] = dump16
        for b in range(_NB):
            fslot = 1 + (b % 3)
            if b >= 3:
                pltpu.make_async_copy(datb.at[0], region.at[idxb.at[0]],
                                      sem).wait()
            for u in range(_CK // 16):
                fv = fidxb[pl.ds(b * _CK + u * 16, 16)]
                idxb[fslot, pl.ds(u * 16, 16)] = jnp.where(
                    fv != _TS, base + fv, _TRASHL)
            pltpu.async_copy(datb.at[0], region.at[idxb.at[fslot]], sem)
        for _u in range(3):
            pltpu.make_async_copy(datb.at[0], region.at[idxb.at[0]],
                                  sem).wait()

    g = lax.rem(s, 4)
    q = s // 4
    for p in range(4):
        @pl.when(s // 4 == p)
        def _():
            scatter_all()

        plsc.subcore_barrier()
        wid_own = (4 * p + g) * _NC + c
        pltpu.sync_copy(region.at[pl.ds(g * _TS + q * gslice, gslice)],
                        wbuf)
        pltpu.sync_copy(
            wbuf, bins_hbm.at[pl.ds(wid_own * _TS + q * gslice, gslice)])
        plsc.subcore_barrier()


def _agg_body(bins_hbm, cnts_hbm, g_hbm, out_hbm, pk, srcb, dstb, rows, zb,
              wb, cbufa, cbufb, agg_sh, bsem, gsem, ssem):
    c = lax.axis_index("c")
    s = lax.axis_index("s")

    z = jnp.zeros((16,), jnp.float32)

    def zz(i, _):
        for t in range(_DP // 16):
            zb[i, pl.ds(t * 16, 16)] = z
        return 0

    lax.fori_loop(0, _ARPT, zz, 0)

    pltpu.sync_copy(cnts_hbm.at[pl.ds((2 * s) * 16, 16)], cbufa)
    pltpu.sync_copy(cnts_hbm.at[pl.ds((2 * s + 1) * 16, 16)], cbufb)


    cva = cbufa[...]
    cvb = cbufb[...]

    for r in range(_NR):
        b = 2 * r + c
        pltpu.sync_copy(zb, agg_sh.at[pl.ds(s * _ARPT, _ARPT)])
        plsc.subcore_barrier()

        ma = jnp.where(c == 0, cva[2 * r], cva[2 * r + 1])
        mb = jnp.where(c == 0, cvb[2 * r], cvb[2 * r + 1])
        mt = ma + mb
        base_a = ((2 * s) * _NB + b) * _CAP
        base_b = ((2 * s + 1) * _NB + b) * _CAP

        def off(j):
            return jnp.where(j < ma, base_a + j * _CK,
                             base_b + (j - ma) * _CK)

        def bcopy(j, slot):
            pltpu.async_copy(bins_hbm.at[pl.ds(off(j), _CK)],
                             pk.at[slot], bsem)

        for k in range(3):
            @pl.when(k < mt)
            def _():
                bcopy(k, k)

        def chunk(j, _):
            slot = lax.rem(j, 4)

            @pl.when(j >= 4)
            def _():
                pltpu.make_async_copy(
                    rows.at[0], agg_sh.at[dstb.at[0]], ssem).wait()

            pltpu.make_async_copy(
                bins_hbm.at[pl.ds(0, _CK)], pk.at[0], bsem).wait()
            for u in range(_CK // 16):
                p = pk[slot, pl.ds(u * 16, 16)]
                srcb[slot, pl.ds(u * 16, 16)] = p & 16383
                dstb[slot, pl.ds(u * 16, 16)] = p >> 14

            @pl.when(j + 3 < mt)
            def _():
                bcopy(j + 3, lax.rem(j + 3, 4))

            pltpu.async_copy(g_hbm.at[srcb.at[slot]], rows.at[slot], gsem)

            @pl.when(j >= 1)
            def _():
                prev = lax.rem(j - 1, 4)
                pltpu.make_async_copy(
                    g_hbm.at[srcb.at[0]], rows.at[0], gsem).wait()
                pltpu.async_copy(rows.at[prev], agg_sh.at[dstb.at[prev]],
                                 ssem, add=True)
            return 0

        lax.fori_loop(0, mt, chunk, 0)

        @pl.when(mt >= 1)
        def _():
            last = lax.rem(mt - 1, 4)
            pltpu.make_async_copy(
                g_hbm.at[srcb.at[0]], rows.at[0], gsem).wait()
            pltpu.async_copy(rows.at[last], agg_sh.at[dstb.at[last]],
                             ssem, add=True)

        def drain(j, _):
            pltpu.make_async_copy(
                rows.at[0], agg_sh.at[dstb.at[0]], ssem).wait()
            return 0

        lax.fori_loop(0, jnp.minimum(mt, 4), drain, 0)

        plsc.subcore_barrier()
        pltpu.sync_copy(agg_sh.at[pl.ds(s * _ARPT, _ARPT)], wb)
        pltpu.sync_copy(wb, out_hbm.at[pl.ds(b * _ACC + s * _ARPT, _ARPT)])
        plsc.subcore_barrier()


@functools.cache
def _sc_kernels():
    mesh = plsc.VectorSubcoreMesh(
        core_axis_name="c", subcore_axis_name="s",
        num_cores=_NC, num_subcores=_NS,
    )
    deg = functools.partial(
        pl.kernel,
        out_type=jax.ShapeDtypeStruct((_NC * _NP,), jnp.float32),
        mesh=mesh,
        scratch_types=[
            pltpu.VMEM((_NCH, _K), jnp.int32),
            pltpu.VMEM((_K,), jnp.float32),
            pltpu.VMEM((_DRPT,), jnp.float32),
            pltpu.VMEM_SHARED((_NP,), jnp.float32),
            pltpu.SemaphoreType.DMA,
        ],
    )(_deg_body)
    binf = functools.partial(
        pl.kernel,
        out_type=jax.ShapeDtypeStruct((_NW * _TS,), jnp.int32),
        mesh=mesh,
        scratch_types=[
            pltpu.VMEM((_EPT,), jnp.int32),
            pltpu.VMEM((_EPT,), jnp.int32),
            pltpu.VMEM((_EPT,), jnp.int32),
            pltpu.VMEM((4, _CK), jnp.int32),
            pltpu.VMEM((4, _CK), jnp.int32),
            pltpu.VMEM((_NB * _CK,), jnp.int32),
            pltpu.VMEM((_GRP * _TS // _NS,), jnp.int32),
            pltpu.VMEM_SHARED((_GRP * _TS + 16,), jnp.int32),
            pltpu.SemaphoreType.DMA,
        ],
    )(_bin_body)
    agg = functools.partial(
        pl.kernel,
        out_type=jax.ShapeDtypeStruct((_OUTR, _DP), jnp.float32),
        mesh=mesh,
        scratch_types=[
            pltpu.VMEM((4, _CK), jnp.int32),
            pltpu.VMEM((4, _CK), jnp.int32),
            pltpu.VMEM((4, _CK), jnp.int32),
            pltpu.VMEM((4, _CK, _DP), jnp.float32),
            pltpu.VMEM((_ARPT, _DP), jnp.float32),
            pltpu.VMEM((_ARPT, _DP), jnp.float32),
            pltpu.VMEM((16,), jnp.int32),
            pltpu.VMEM((16,), jnp.int32),
            pltpu.VMEM_SHARED((_ACC, _DP), jnp.float32),
            pltpu.SemaphoreType.DMA,
            pltpu.SemaphoreType.DMA,
            pltpu.SemaphoreType.DMA,
        ],
    )(_agg_body)
    return deg, binf, agg


def _gcn_in(aggp_ref):
    parts = []
    done = 0
    for b in range(_NB):
        n = min(_RNG, _N - done)
        if n > 0:
            parts.append(aggp_ref[b * _ACC:b * _ACC + n, :_DH])
        done += n
    return jnp.concatenate(parts, axis=0)


def _pos_body(dst3_ref, pos_ref, fill_ref, cnts_ref):
    d = dst3_ref[...]
    bkt = (d * 18894) >> 25
    jrow = lax.broadcasted_iota(jnp.int32, (625, 625), 0)
    icol = lax.broadcasted_iota(jnp.int32, (625, 625), 1)
    triv = (jrow < icol).astype(jnp.float32)
    j16 = lax.broadcasted_iota(jnp.int32, (16, 16), 0)
    i16 = lax.broadcasted_iota(jnp.int32, (16, 16), 1)
    tril = (j16 < i16).astype(jnp.float32)
    t = pl.program_id(0)
    bidx16 = lax.broadcasted_iota(jnp.int32, (1, 1, 16), 2)
    kfid = lax.broadcasted_iota(jnp.int32, (1, _CK), 1)
    pos = jnp.zeros((1, 625, 16), jnp.int32)
    cnts = jnp.zeros((1, 1, 16), jnp.int32)
    fills = []
    for b in range(_NB):
        oh = (bkt == b).astype(jnp.float32)
        vcnt = jnp.sum(oh, axis=2)
        vpre = lax.dot_general(vcnt, triv, (((1,), (0,)), ((), ())),
                               preferred_element_type=jnp.float32)
        lpre = lax.dot_general(oh, tril, (((2,), (0,)), ((), ())),
                               preferred_element_type=jnp.float32)
        rank = (vpre[:, :, None] + lpre).astype(jnp.int32)
        pos = pos + jnp.where(bkt == b, b * _CAP + rank, 0)
        tot = (vpre[:, 624] + vcnt[:, 624]).astype(jnp.int32)
        ncb = (tot + _CK - 1) // _CK
        cnts = jnp.where(bidx16 == b, ncb[:, None, None], cnts)
        nfill = ncb * _CK - tot
        fp = jnp.where(kfid < nfill[:, None],
                       b * _CAP + tot[:, None] + kfid,
                       _TS)
        fills.append(fp[:, None, :])
    pos_ref[...] = pos + t * 0
    fill_ref[...] = jnp.concatenate(fills, axis=1)
    cnts_ref[...] = cnts


_pos_call = pl.pallas_call(
    _pos_body,
    grid=(_NW,),
    in_specs=[pl.BlockSpec((1, 625, 16), lambda t: (t, 0, 0))],
    out_specs=[
        pl.BlockSpec((1, 625, 16), lambda t: (t, 0, 0)),
        pl.BlockSpec((1, _NB, _CK), lambda t: (t, 0, 0)),
        pl.BlockSpec((1, 1, 16), lambda t: (t, 0, 0)),
    ],
    out_shape=[
        jax.ShapeDtypeStruct((_NW, 625, 16), jnp.int32),
        jax.ShapeDtypeStruct((_NW, _NB, _CK), jnp.int32),
        jax.ShapeDtypeStruct((_NW, 1, 16), jnp.int32),
    ],
)


def _prep_body(degp_ref, x_ref, w1_ref, g1_ref, dinv_ref):
    deg = degp_ref[0, :_N] + degp_ref[1, :_N] + 1.0
    dinv = lax.rsqrt(deg).reshape(_N, 1)
    h = jnp.dot(x_ref[...], w1_ref[...], preferred_element_type=jnp.float32)
    g1_ref[...] = h * dinv
    dinv_ref[...] = dinv


def _mid_body(aggp_ref, g_ref, dinv_ref, b1_ref, w2_ref, g2_ref):
    agg = _gcn_in(aggp_ref) + g_ref[:, :_DH]
    h = jnp.maximum(agg * dinv_ref[...] + b1_ref[...], 0.0)
    g2_ref[...] = jnp.dot(
        h, w2_ref[...], preferred_element_type=jnp.float32) * dinv_ref[...]


def _fin_body(aggp_ref, g2_ref, dinv_ref, b2_ref, batch_ref, wfc_ref, bfc_ref,
              out_ref):
    agg = _gcn_in(aggp_ref) + g2_ref[:, :_DH]
    h = jnp.maximum(agg * dinv_ref[...] + b2_ref[...], 0.0)
    onehot = (batch_ref[...] == lax.broadcasted_iota(
        jnp.int32, (1, _G), 1)).astype(jnp.float32)
    pooled = lax.dot_general(onehot, h, (((0,), (0,)), ((), ())),
                             preferred_element_type=jnp.float32)
    cnt = lax.dot_general(onehot, jnp.ones((_N, 1), jnp.float32),
                          (((0,), (0,)), ((), ())),
                          preferred_element_type=jnp.float32)
    pooled = pooled / jnp.maximum(cnt, 1.0)
    out_ref[...] = jnp.dot(
        pooled, wfc_ref[...], preferred_element_type=jnp.float32) + bfc_ref[...]


_prep_call = pl.pallas_call(
    _prep_body,
    out_shape=[
        jax.ShapeDtypeStruct((_N, _DP), jnp.float32),
        jax.ShapeDtypeStruct((_N, 1), jnp.float32),
    ],
)

_mid_call = pl.pallas_call(
    _mid_body,
    out_shape=jax.ShapeDtypeStruct((_N, _DP), jnp.float32),
)

_fin_call = pl.pallas_call(
    _fin_body,
    out_shape=jax.ShapeDtypeStruct((_G, 1), jnp.float32),
)


def kernel(x, edge_index, batch, W1, b1, W2, b2, Wfc, bfc):
    _deg_kernel, _bin_kernel, _agg_kernel = _sc_kernels()
    src1d = edge_index[0].reshape(_E)
    dst1d = edge_index[1].reshape(_E)
    dst_deg = edge_index[1].reshape(_NW, _NCH, _K)
    w1p = jnp.pad(W1, ((0, 0), (0, _DP - _DH)))
    w2p = jnp.pad(W2, ((0, 0), (0, _DP - _DH)))

    dst3 = edge_index[1].reshape(_NW, 625, 16)
    post, fill3, cnts2d = _pos_call(dst3)
    pos1d = post.reshape(_E)
    fill1d = fill3.reshape(_NW * _NB * _CK)
    cnts = cnts2d.reshape(_NW * 16)

    degp = _deg_kernel(dst_deg).reshape(_NC, _NP)
    bins = _bin_kernel(src1d, dst1d, pos1d, fill1d)
    g1, dinv = _prep_call(degp, x, w1p)
    aggp1 = _agg_kernel(bins, cnts, g1)
    g2 = _mid_call(aggp1, g1, dinv, b1.reshape(1, _DH), w2p)
    aggp2 = _agg_kernel(bins, cnts, g2)
    out = _fin_call(aggp2, g2, dinv, b2.reshape(1, _DH), batch.reshape(_N, 1),
                    Wfc, bfc.reshape(1, 1))
    return out
